# Initial kernel scaffold; baseline (speedup 1.0000x reference)
import jax
import jax.numpy as jnp
from jax.experimental import pallas as pl


def kernel(input_ids, token_type_ids, position_ids, attention_mask, W_word, W_pos, W_type, gamma, beta):
    raise NotImplementedError("write your pallas kernel here")


# speedup vs baseline: 3.4601x; 3.4601x over previous
import functools

import jax
import jax.numpy as jnp
from jax import lax
from jax.experimental import pallas as pl
from jax.experimental.pallas import tpu as pltpu
from jax.experimental.pallas import tpu_sc as plsc

VOCAB = 100000
D = 128
POS_V = 512
TYPE_V = 2
EPS = 1e-06

NC = 2
NS = 16
NW = NC * NS
L = 16

N_TOK = 1024 * 200
PER_W = N_TOK // NW
C = 128
CHUNKS = PER_W // C


def _emb_body(ids_hbm, pos_hbm, tid_hbm, wword_hbm, wpos_hbm, wtype_hbm,
              gam_hbm, bet_hbm, out_hbm,
              idxw_v, idxp_v, idxt_v, rows_v, wpos_v, wtype_v, gam_v, bet_v,
              sem):
    wid = lax.axis_index("s") * NC + lax.axis_index("c")
    base_w = wid * PER_W

    pltpu.sync_copy(wpos_hbm, wpos_v)
    pltpu.sync_copy(wtype_hbm, wtype_v)
    pltpu.sync_copy(gam_hbm, gam_v)
    pltpu.sync_copy(bet_hbm, bet_v)

    iota = lax.iota(jnp.int32, L)
    t0 = [wtype_v[0, pl.ds(L * j, L)] for j in range(D // L)]
    td = [wtype_v[1, pl.ds(L * j, L)] - t0[j] for j in range(D // L)]
    gv = [gam_v[pl.ds(L * j, L)] for j in range(D // L)]
    bv = [bet_v[pl.ds(L * j, L)] for j in range(D // L)]

    def chunk_body(c, carry):
        base = base_w + c * C
        pltpu.sync_copy(ids_hbm.at[pl.ds(base, C)], idxw_v)
        pltpu.sync_copy(pos_hbm.at[pl.ds(base, C)], idxp_v)
        pltpu.sync_copy(tid_hbm.at[pl.ds(base, C)], idxt_v)
        pltpu.async_copy(wword_hbm.at[idxw_v], rows_v, sem).wait()

        def tok(t, tc):
            tsp = jnp.full((L,), t, jnp.int32)
            psp = plsc.load_gather(idxp_v, [tsp])
            ttf = plsc.load_gather(idxt_v, [tsp]).astype(jnp.float32)
            acc = jnp.zeros((L,), jnp.float32)
            acc2 = jnp.zeros((L,), jnp.float32)
            vs = []
            for j in range(D // L):
                w = rows_v[t, pl.ds(L * j, L)]
                p = plsc.load_gather(wpos_v, [psp, iota + (L * j)])
                v = w + p + t0[j] + ttf * td[j]
                acc = acc + v
                acc2 = acc2 + v * v
                vs.append(v)
            s = jnp.sum(acc)
            s2 = jnp.sum(acc2)
            mean = s * (1.0 / D)
            var = s2 * (1.0 / D) - mean * mean + EPS
            var_v = jnp.full((L,), var, jnp.float32)
            mean_v = jnp.full((L,), mean, jnp.float32)
            i = plsc.bitcast(var_v, jnp.int32)
            y = plsc.bitcast(jnp.int32(0x5F3759DF) - (i >> 1), jnp.float32)
            for _ in range(3):
                y = y * (1.5 - 0.5 * var_v * y * y)
            for j in range(D // L):
                rows_v[t, pl.ds(L * j, L)] = (vs[j] - mean_v) * (y * gv[j]) + bv[j]
            return tc

        lax.fori_loop(0, C, tok, 0, unroll=False)
        pltpu.sync_copy(rows_v, out_hbm.at[pl.ds(base, C)])
        return carry

    lax.fori_loop(0, CHUNKS, chunk_body, 0, unroll=False)


def kernel(input_ids, token_type_ids, position_ids, attention_mask,
           W_word, W_pos, W_type, gamma, beta):
    del attention_mask
    ids = input_ids.reshape(-1).astype(jnp.int32)
    pos = position_ids.reshape(-1).astype(jnp.int32)
    tid = token_type_ids.reshape(-1).astype(jnp.int32)

    mesh = plsc.VectorSubcoreMesh(core_axis_name="c", subcore_axis_name="s",
                                  num_cores=NC, num_subcores=NS)
    run = pl.kernel(
        _emb_body,
        out_type=jax.ShapeDtypeStruct((N_TOK, D), jnp.float32),
        mesh=mesh,
        scratch_types=[
            pltpu.VMEM((C,), jnp.int32),
            pltpu.VMEM((C,), jnp.int32),
            pltpu.VMEM((C,), jnp.int32),
            pltpu.VMEM((C, D), jnp.float32),
            pltpu.VMEM((POS_V, D), jnp.float32),
            pltpu.VMEM((TYPE_V, D), jnp.float32),
            pltpu.VMEM((D,), jnp.float32),
            pltpu.VMEM((D,), jnp.float32),
            pltpu.SemaphoreType.DMA,
        ],
        compiler_params=pltpu.CompilerParams(needs_layout_passes=False),
    )
    out = run(ids, pos, tid,
              W_word.astype(jnp.float32), W_pos.astype(jnp.float32),
              W_type.astype(jnp.float32),
              gamma.astype(jnp.float32), beta.astype(jnp.float32))
    return out.reshape(input_ids.shape + (D,))


# speedup vs baseline: 3.4861x; 1.0075x over previous
import functools

import jax
import jax.numpy as jnp
from jax import lax
from jax.experimental import pallas as pl
from jax.experimental.pallas import tpu as pltpu
from jax.experimental.pallas import tpu_sc as plsc

VOCAB = 100000
D = 128
POS_V = 512
TYPE_V = 2
EPS = 1e-06

NC = 2
NS = 16
NW = NC * NS
L = 16

N_TOK = 1024 * 200
PER_W = N_TOK // NW
C = 128
CHUNKS = PER_W // C


def _emb_body(ids_hbm, pos_hbm, tid_hbm, wword_hbm, wpos_hbm, wtype_hbm,
              gam_hbm, bet_hbm, out_hbm,
              idxw_v, idxp_v, idxt_v, rows_v, wpos_v, wtype_v, gam_v, bet_v,
              sem):
    wid = lax.axis_index("s") * NC + lax.axis_index("c")
    base_w = wid * PER_W

    pltpu.sync_copy(wpos_hbm, wpos_v)
    pltpu.sync_copy(wtype_hbm, wtype_v)
    pltpu.sync_copy(gam_hbm, gam_v)
    pltpu.sync_copy(bet_hbm, bet_v)

    iota = lax.iota(jnp.int32, L)
    col_idx = [iota + (L * j) for j in range(D // L)]
    t0 = [wtype_v[0, pl.ds(L * j, L)] for j in range(D // L)]
    td = [wtype_v[1, pl.ds(L * j, L)] - t0[j] for j in range(D // L)]
    gv = [gam_v[pl.ds(L * j, L)] for j in range(D // L)]
    bv = [bet_v[pl.ds(L * j, L)] for j in range(D // L)]

    def chunk_body(c, carry):
        base = base_w + c * C
        pltpu.sync_copy(ids_hbm.at[pl.ds(base, C)], idxw_v)
        pltpu.sync_copy(pos_hbm.at[pl.ds(base, C)], idxp_v)
        pltpu.sync_copy(tid_hbm.at[pl.ds(base, C)], idxt_v)
        pltpu.async_copy(wword_hbm.at[idxw_v], rows_v, sem).wait()

        def tok(t, tc):
            tsp = jnp.full((L,), t, jnp.int32)
            psp = plsc.load_gather(idxp_v, [tsp])
            ttf = plsc.load_gather(idxt_v, [tsp]).astype(jnp.float32)
            acc = jnp.zeros((L,), jnp.float32)
            acc2 = jnp.zeros((L,), jnp.float32)
            vs = []
            for j in range(D // L):
                w = rows_v[t, pl.ds(L * j, L)]
                p = plsc.load_gather(wpos_v, [psp, col_idx[j]])
                v = w + p + t0[j] + ttf * td[j]
                acc = acc + v
                acc2 = acc2 + v * v
                vs.append(v)
            s = jnp.sum(acc)
            s2 = jnp.sum(acc2)
            mean = s * (1.0 / D)
            var = s2 * (1.0 / D) - mean * mean + EPS
            var_v = jnp.full((L,), var, jnp.float32)
            mean_v = jnp.full((L,), mean, jnp.float32)
            i = plsc.bitcast(var_v, jnp.int32)
            y = plsc.bitcast(jnp.int32(0x5F3759DF) - (i >> 1), jnp.float32)
            for _ in range(3):
                y = y * (1.5 - 0.5 * var_v * y * y)
            for j in range(D // L):
                rows_v[t, pl.ds(L * j, L)] = (vs[j] - mean_v) * (y * gv[j]) + bv[j]
            return tc

        lax.fori_loop(0, C, tok, 0, unroll=2)
        pltpu.sync_copy(rows_v, out_hbm.at[pl.ds(base, C)])
        return carry

    lax.fori_loop(0, CHUNKS, chunk_body, 0, unroll=False)


def kernel(input_ids, token_type_ids, position_ids, attention_mask,
           W_word, W_pos, W_type, gamma, beta):
    del attention_mask
    ids = input_ids.reshape(-1).astype(jnp.int32)
    pos = position_ids.reshape(-1).astype(jnp.int32)
    tid = token_type_ids.reshape(-1).astype(jnp.int32)

    mesh = plsc.VectorSubcoreMesh(core_axis_name="c", subcore_axis_name="s",
                                  num_cores=NC, num_subcores=NS)
    run = pl.kernel(
        _emb_body,
        out_type=jax.ShapeDtypeStruct((N_TOK, D), jnp.float32),
        mesh=mesh,
        scratch_types=[
            pltpu.VMEM((C,), jnp.int32),
            pltpu.VMEM((C,), jnp.int32),
            pltpu.VMEM((C,), jnp.int32),
            pltpu.VMEM((C, D), jnp.float32),
            pltpu.VMEM((POS_V, D), jnp.float32),
            pltpu.VMEM((TYPE_V, D), jnp.float32),
            pltpu.VMEM((D,), jnp.float32),
            pltpu.VMEM((D,), jnp.float32),
            pltpu.SemaphoreType.DMA,
        ],
        compiler_params=pltpu.CompilerParams(needs_layout_passes=False),
    )
    out = run(ids, pos, tid,
              W_word.astype(jnp.float32), W_pos.astype(jnp.float32),
              W_type.astype(jnp.float32),
              gamma.astype(jnp.float32), beta.astype(jnp.float32))
    return out.reshape(input_ids.shape + (D,))


# speedup vs baseline: 4.8653x; 1.3956x over previous
import functools

import jax
import jax.numpy as jnp
from jax import lax
from jax.experimental import pallas as pl
from jax.experimental.pallas import tpu as pltpu
from jax.experimental.pallas import tpu_sc as plsc

VOCAB = 100000
D = 128
POS_V = 512
TYPE_V = 2
EPS = 1e-06

NC = 2
NS = 16
NW = NC * NS
L = 16

N_TOK = 1024 * 200
PER_W = N_TOK // NW
C = 128
CHUNKS = PER_W // C


def _emb_body(ids_hbm, pos_hbm, tid_hbm, wword_hbm, wpos_hbm, wtype_hbm,
              gam_hbm, bet_hbm, out_hbm,
              idxw_v, idxp_v, idxt_v, rows_v, out_v, wpos_v, wtype_v, gam_v,
              bet_v, sem):
    wid = lax.axis_index("s") * NC + lax.axis_index("c")
    base_w = wid * PER_W

    pltpu.sync_copy(wpos_hbm, wpos_v)
    pltpu.sync_copy(wtype_hbm, wtype_v)
    pltpu.sync_copy(gam_hbm, gam_v)
    pltpu.sync_copy(bet_hbm, bet_v)

    iota = lax.iota(jnp.int32, L)
    col_idx = [iota + (L * j) for j in range(D // L)]
    t0 = [wtype_v[0, pl.ds(L * j, L)] for j in range(D // L)]
    td = [wtype_v[1, pl.ds(L * j, L)] - t0[j] for j in range(D // L)]
    gv = [gam_v[pl.ds(L * j, L)] for j in range(D // L)]
    bv = [bet_v[pl.ds(L * j, L)] for j in range(D // L)]

    def chunk_body(c, carry):
        base = base_w + c * C
        pltpu.sync_copy(ids_hbm.at[pl.ds(base, C)], idxw_v)
        pltpu.sync_copy(pos_hbm.at[pl.ds(base, C)], idxp_v)
        pltpu.sync_copy(tid_hbm.at[pl.ds(base, C)], idxt_v)
        pltpu.async_copy(wword_hbm.at[idxw_v], rows_v, sem).wait()

        @plsc.parallel_loop(0, C, 1, unroll=2)
        def tok(t):
            tsp = jnp.full((L,), t, jnp.int32)
            psp = plsc.load_gather(idxp_v, [tsp])
            ttf = plsc.load_gather(idxt_v, [tsp]).astype(jnp.float32)
            acc = jnp.zeros((L,), jnp.float32)
            acc2 = jnp.zeros((L,), jnp.float32)
            vs = []
            for j in range(D // L):
                w = rows_v[t, pl.ds(L * j, L)]
                p = plsc.load_gather(wpos_v, [psp, col_idx[j]])
                v = w + p + t0[j] + ttf * td[j]
                acc = acc + v
                acc2 = acc2 + v * v
                vs.append(v)
            s = jnp.sum(acc)
            s2 = jnp.sum(acc2)
            mean = s * (1.0 / D)
            var = s2 * (1.0 / D) - mean * mean + EPS
            var_v = jnp.full((L,), var, jnp.float32)
            mean_v = jnp.full((L,), mean, jnp.float32)
            i = plsc.bitcast(var_v, jnp.int32)
            y = plsc.bitcast(jnp.int32(0x5F3759DF) - (i >> 1), jnp.float32)
            for _ in range(3):
                y = y * (1.5 - 0.5 * var_v * y * y)
            for j in range(D // L):
                out_v[t, pl.ds(L * j, L)] = (vs[j] - mean_v) * (y * gv[j]) + bv[j]

        pltpu.sync_copy(out_v, out_hbm.at[pl.ds(base, C)])
        return carry

    lax.fori_loop(0, CHUNKS, chunk_body, 0, unroll=False)


def kernel(input_ids, token_type_ids, position_ids, attention_mask,
           W_word, W_pos, W_type, gamma, beta):
    del attention_mask
    ids = input_ids.reshape(-1).astype(jnp.int32)
    pos = position_ids.reshape(-1).astype(jnp.int32)
    tid = token_type_ids.reshape(-1).astype(jnp.int32)

    mesh = plsc.VectorSubcoreMesh(core_axis_name="c", subcore_axis_name="s",
                                  num_cores=NC, num_subcores=NS)
    run = pl.kernel(
        _emb_body,
        out_type=jax.ShapeDtypeStruct((N_TOK, D), jnp.float32),
        mesh=mesh,
        scratch_types=[
            pltpu.VMEM((C,), jnp.int32),
            pltpu.VMEM((C,), jnp.int32),
            pltpu.VMEM((C,), jnp.int32),
            pltpu.VMEM((C, D), jnp.float32),
            pltpu.VMEM((C, D), jnp.float32),
            pltpu.VMEM((POS_V, D), jnp.float32),
            pltpu.VMEM((TYPE_V, D), jnp.float32),
            pltpu.VMEM((D,), jnp.float32),
            pltpu.VMEM((D,), jnp.float32),
            pltpu.SemaphoreType.DMA,
        ],
        compiler_params=pltpu.CompilerParams(needs_layout_passes=False),
    )
    out = run(ids, pos, tid,
              W_word.astype(jnp.float32), W_pos.astype(jnp.float32),
              W_type.astype(jnp.float32),
              gamma.astype(jnp.float32), beta.astype(jnp.float32))
    return out.reshape(input_ids.shape + (D,))
